# baseline (speedup 1.0000x reference)
import functools

import jax
import jax.numpy as jnp
from jax import lax
from jax.experimental import pallas as pl
from jax.experimental.pallas import tpu as pltpu
from jax.experimental.pallas import tpu_sc as plsc

N_FIELDS = 26
VOCAB = 100000
DIM = 16
NC = 2
NS = 16
NW = NC * NS
TROWS = N_FIELDS * DIM
RPW = TROWS // NW
BCH = 4096
NCH = 4
QLEN = 196 * 128
QTAIL = 781 * 128 - 3 * QLEN
VMAIN = 781 * 128
TBASE = VOCAB - 256


def _lookup_body(idx_hbm, tab_hbm, tails_hbm, out_hbm, trow_v, idx_v, grow_v,
                 tails_v, sem_t, sem_i, sem_tl, sem_w0, sem_w1):
    wid = lax.axis_index("s") * NC + lax.axis_index("c")
    sem_w = (sem_w0, sem_w1)
    row0 = wid * RPW
    blk0 = pl.multiple_of(row0 // 8 * 8, 8)

    pltpu.sync_copy(tails_hbm.at[pl.ds(blk0, 24)], tails_v)

    def row_streams(row, start):
        for q in range(4):
            ln = QLEN if q < 3 else QTAIL
            src = tab_hbm.at[row, pl.ds(q * QLEN, ln)]
            dst = trow_v.at[pl.ds(q * QLEN, ln)]
            if start:
                pltpu.async_copy(src, dst, sem_t)
            else:
                pltpu.make_async_copy(src, dst, sem_t).wait()

    def row_loop(r, prev_f):
        row = row0 + r
        f = row // DIM
        row_streams(row, start=True)

        @pl.when(f != prev_f)
        def _():
            pltpu.sync_copy(idx_hbm.at[f], idx_v)

        row_streams(row, start=False)
        jvec = jnp.zeros((16,), jnp.int32) + (row - blk0)

        for c in range(NCH):
            s = c % 2
            if c >= 2:
                pltpu.make_async_copy(
                    grow_v.at[s], out_hbm.at[row, pl.ds(0, BCH)], sem_w[s]
                ).wait()
            else:
                @pl.when(r > 0)
                def _():
                    pltpu.make_async_copy(
                        grow_v.at[s], out_hbm.at[row, pl.ds(0, BCH)], sem_w[s]
                    ).wait()

            def g(i, carry3):
                iv = idx_v[pl.ds(c * BCH + i * 16, 16)]
                vals = plsc.load_gather(trow_v, [iv])
                tmask = iv >= TBASE
                tvals = plsc.load_gather(
                    tails_v, [jvec, iv - TBASE], mask=tmask
                )
                grow_v[s, pl.ds(i * 16, 16)] = jnp.where(tmask, tvals, vals)
                return carry3

            lax.fori_loop(0, BCH // 16, g, 0, unroll=4)
            pltpu.async_copy(
                grow_v.at[s], out_hbm.at[row, pl.ds(c * BCH, BCH)], sem_w[s]
            )
        return f

    lax.fori_loop(0, RPW, row_loop, -1)

    last_row = row0 + RPW - 1
    for s in range(2):
        pltpu.make_async_copy(
            grow_v.at[s], out_hbm.at[last_row, pl.ds(0, BCH)], sem_w[s]
        ).wait()


def kernel(indices, tables):
    batch = indices.shape[0]

    idx_t = indices.T
    tab_t = jnp.transpose(tables, (0, 2, 1)).reshape(TROWS, VOCAB)
    tails_t = jnp.transpose(tables[:, TBASE:, :], (0, 2, 1)).reshape(
        TROWS, 256
    )
    tails_t = jnp.concatenate(
        [tails_t, jnp.zeros((16, 256), jnp.float32)], axis=0
    )

    mesh = plsc.VectorSubcoreMesh(core_axis_name="c", subcore_axis_name="s")
    lookup = functools.partial(
        pl.kernel,
        out_type=jax.ShapeDtypeStruct((TROWS, batch), jnp.float32),
        mesh=mesh,
        scratch_types=[
            pltpu.VMEM((VOCAB,), jnp.float32),
            pltpu.VMEM((batch,), jnp.int32),
            pltpu.VMEM((2, BCH), jnp.float32),
            pltpu.VMEM((24, 256), jnp.float32),
            pltpu.SemaphoreType.DMA,
            pltpu.SemaphoreType.DMA,
            pltpu.SemaphoreType.DMA,
            pltpu.SemaphoreType.DMA,
            pltpu.SemaphoreType.DMA,
        ],
        compiler_params=pltpu.CompilerParams(
            use_tc_tiling_on_sc=True, needs_layout_passes=False
        ),
    )(_lookup_body)

    out_t = lookup(idx_t, tab_t, tails_t)
    return out_t.T


# speedup vs baseline: 1.2384x; 1.2384x over previous
import functools

import jax
import jax.numpy as jnp
from jax import lax
from jax.experimental import pallas as pl
from jax.experimental.pallas import tpu as pltpu
from jax.experimental.pallas import tpu_sc as plsc

N_FIELDS = 26
VOCAB = 100000
DIM = 16
NC = 2
NS = 16
NW = NC * NS
TROWS = N_FIELDS * DIM
RPW = TROWS // NW
BCH = 4096
NCH = 4
QLEN = 196 * 128
QTAIL = 781 * 128 - 3 * QLEN
VMAIN = 781 * 128
TBASE = VOCAB - 256


def _lookup_body(idx_hbm, tab_hbm, tails_hbm, out_hbm, trow_v, idx_v, grow_v,
                 tails_v, sem_t, sem_i, sem_tl, sem_w0, sem_w1):
    wid = lax.axis_index("s") * NC + lax.axis_index("c")
    sem_w = (sem_w0, sem_w1)
    row0 = wid * RPW
    blk0 = pl.multiple_of(row0 // 8 * 8, 8)

    pltpu.sync_copy(tails_hbm.at[pl.ds(blk0, 24)], tails_v)

    HLEN0 = 391 * 128
    HLEN1 = VMAIN - HLEN0

    def row_streams(row, start):
        for off, ln in ((0, HLEN0), (HLEN0, HLEN1)):
            src = tab_hbm.at[row, pl.ds(off, ln)]
            dst = trow_v.at[pl.ds(off, ln)]
            if start:
                pltpu.async_copy(src, dst, sem_t)
            else:
                pltpu.make_async_copy(src, dst, sem_t).wait()

    def row_loop(r, prev_f):
        row = row0 + r
        f = row // DIM
        row_streams(row, start=True)

        @pl.when(f != prev_f)
        def _():
            pltpu.sync_copy(idx_hbm.at[f], idx_v)

        row_streams(row, start=False)
        jrow = row - blk0

        def patch(t, carry2):
            trow_v[pl.ds(TBASE + t * 16, 16)] = tails_v[
                jrow, pl.ds(t * 16, 16)
            ]
            return carry2

        lax.fori_loop(0, 16, patch, 0, unroll=8)

        for c in range(NCH):
            s = c % 2
            if c >= 2:
                pltpu.make_async_copy(
                    grow_v.at[s], out_hbm.at[row, pl.ds(0, BCH)], sem_w[s]
                ).wait()
            else:
                @pl.when(r > 0)
                def _():
                    pltpu.make_async_copy(
                        grow_v.at[s], out_hbm.at[row, pl.ds(0, BCH)], sem_w[s]
                    ).wait()

            def g(i, carry3):
                iv = idx_v[pl.ds(c * BCH + i * 16, 16)]
                grow_v[s, pl.ds(i * 16, 16)] = plsc.load_gather(trow_v, [iv])
                return carry3

            lax.fori_loop(0, BCH // 16, g, 0, unroll=4)
            pltpu.async_copy(
                grow_v.at[s], out_hbm.at[row, pl.ds(c * BCH, BCH)], sem_w[s]
            )
        return f

    lax.fori_loop(0, RPW, row_loop, -1)

    last_row = row0 + RPW - 1
    for s in range(2):
        pltpu.make_async_copy(
            grow_v.at[s], out_hbm.at[last_row, pl.ds(0, BCH)], sem_w[s]
        ).wait()


def kernel(indices, tables):
    batch = indices.shape[0]

    idx_t = indices.T
    tab_t = jnp.transpose(tables, (0, 2, 1)).reshape(TROWS, VOCAB)
    tails_t = jnp.transpose(tables[:, TBASE:, :], (0, 2, 1)).reshape(
        TROWS, 256
    )
    tails_t = jnp.concatenate(
        [tails_t, jnp.zeros((16, 256), jnp.float32)], axis=0
    )

    mesh = plsc.VectorSubcoreMesh(core_axis_name="c", subcore_axis_name="s")
    lookup = functools.partial(
        pl.kernel,
        out_type=jax.ShapeDtypeStruct((TROWS, batch), jnp.float32),
        mesh=mesh,
        scratch_types=[
            pltpu.VMEM((VOCAB,), jnp.float32),
            pltpu.VMEM((batch,), jnp.int32),
            pltpu.VMEM((2, BCH), jnp.float32),
            pltpu.VMEM((24, 256), jnp.float32),
            pltpu.SemaphoreType.DMA,
            pltpu.SemaphoreType.DMA,
            pltpu.SemaphoreType.DMA,
            pltpu.SemaphoreType.DMA,
            pltpu.SemaphoreType.DMA,
        ],
        compiler_params=pltpu.CompilerParams(
            use_tc_tiling_on_sc=True, needs_layout_passes=False
        ),
    )(_lookup_body)

    out_t = lookup(idx_t, tab_t, tails_t)
    return out_t.T


# speedup vs baseline: 1.2509x; 1.0101x over previous
import functools

import jax
import jax.numpy as jnp
from jax import lax
from jax.experimental import pallas as pl
from jax.experimental.pallas import tpu as pltpu
from jax.experimental.pallas import tpu_sc as plsc

N_FIELDS = 26
VOCAB = 100000
DIM = 16
NC = 2
NS = 16
NW = NC * NS
TROWS = N_FIELDS * DIM
RPW = TROWS // NW
BCH = 4096
NCH = 4


def _lookup_body(idx_hbm, tab_hbm, out_hbm, trow_v, idx_v, grow_v,
                 sem_t, sem_w0, sem_w1):
    wid = lax.axis_index("s") * NC + lax.axis_index("c")
    sem_w = (sem_w0, sem_w1)

    def row_loop(r, prev_f):
        row = wid * RPW + r
        f = row // DIM
        d = lax.rem(row, DIM)

        pltpu.async_copy(tab_hbm.at[f, d], trow_v, sem_t)

        @pl.when(f != prev_f)
        def _():
            pltpu.sync_copy(idx_hbm.at[f], idx_v)

        pltpu.make_async_copy(tab_hbm.at[f, d], trow_v, sem_t).wait()

        for c in range(NCH):
            s = c % 2
            if c >= 2:
                pltpu.make_async_copy(
                    grow_v.at[s], out_hbm.at[row, pl.ds(0, BCH)], sem_w[s]
                ).wait()
            else:
                @pl.when(r > 0)
                def _():
                    pltpu.make_async_copy(
                        grow_v.at[s], out_hbm.at[row, pl.ds(0, BCH)], sem_w[s]
                    ).wait()

            def g(i, carry3):
                iv = idx_v[pl.ds(c * BCH + i * 16, 16)]
                grow_v[s, pl.ds(i * 16, 16)] = plsc.load_gather(trow_v, [iv])
                return carry3

            lax.fori_loop(0, BCH // 16, g, 0, unroll=4)
            pltpu.async_copy(
                grow_v.at[s], out_hbm.at[row, pl.ds(c * BCH, BCH)], sem_w[s]
            )
        return f

    last_f = lax.fori_loop(0, RPW, row_loop, -1)
    last_row = wid * RPW + RPW - 1
    for s in range(2):
        pltpu.make_async_copy(
            grow_v.at[s], out_hbm.at[last_row, pl.ds(0, BCH)], sem_w[s]
        ).wait()


def kernel(indices, tables):
    batch = indices.shape[0]

    idx_t = indices.T
    tab_t = jnp.transpose(tables, (0, 2, 1))

    mesh = plsc.VectorSubcoreMesh(core_axis_name="c", subcore_axis_name="s")
    lookup = functools.partial(
        pl.kernel,
        out_type=jax.ShapeDtypeStruct((TROWS, batch), jnp.float32),
        mesh=mesh,
        scratch_types=[
            pltpu.VMEM((VOCAB,), jnp.float32),
            pltpu.VMEM((batch,), jnp.int32),
            pltpu.VMEM((2, BCH), jnp.float32),
            pltpu.SemaphoreType.DMA,
            pltpu.SemaphoreType.DMA,
            pltpu.SemaphoreType.DMA,
        ],
        compiler_params=pltpu.CompilerParams(
            use_tc_tiling_on_sc=True, needs_layout_passes=False
        ),
    )(_lookup_body)

    out_t = lookup(idx_t, tab_t)
    return out_t.T


# speedup vs baseline: 1.2602x; 1.0075x over previous
import functools

import jax
import jax.numpy as jnp
from jax import lax
from jax.experimental import pallas as pl
from jax.experimental.pallas import tpu as pltpu
from jax.experimental.pallas import tpu_sc as plsc

N_FIELDS = 26
VOCAB = 100000
DIM = 16
NC = 2
NS = 16
NW = NC * NS
TROWS = N_FIELDS * DIM
RPW = TROWS // NW
BCH = 4096
NCH = 4


def _lookup_body(idx_hbm, tab_hbm, out_hbm, trow_v, idx_v, grow_v,
                 sem_t, sem_w0, sem_w1):
    wid = lax.axis_index("s") * NC + lax.axis_index("c")
    sem_w = (sem_w0, sem_w1)

    def row_loop(r, prev_f):
        row = wid * RPW + r
        f = row // DIM
        d = lax.rem(row, DIM)

        pltpu.async_copy(tab_hbm.at[f, d], trow_v, sem_t)

        @pl.when(f != prev_f)
        def _():
            pltpu.sync_copy(idx_hbm.at[f], idx_v)

        pltpu.make_async_copy(tab_hbm.at[f, d], trow_v, sem_t).wait()

        for c in range(NCH):
            s = c % 2
            if c >= 2:
                pltpu.make_async_copy(
                    grow_v.at[s], out_hbm.at[row, pl.ds(0, BCH)], sem_w[s]
                ).wait()
            else:
                @pl.when(r > 0)
                def _():
                    pltpu.make_async_copy(
                        grow_v.at[s], out_hbm.at[row, pl.ds(0, BCH)], sem_w[s]
                    ).wait()

            def g(i, carry3):
                iv = idx_v[pl.ds(c * BCH + i * 16, 16)]
                grow_v[s, pl.ds(i * 16, 16)] = plsc.load_gather(trow_v, [iv])
                return carry3

            lax.fori_loop(0, BCH // 16, g, 0, unroll=8)
            pltpu.async_copy(
                grow_v.at[s], out_hbm.at[row, pl.ds(c * BCH, BCH)], sem_w[s]
            )
        return f

    last_f = lax.fori_loop(0, RPW, row_loop, -1)
    last_row = wid * RPW + RPW - 1
    for s in range(2):
        pltpu.make_async_copy(
            grow_v.at[s], out_hbm.at[last_row, pl.ds(0, BCH)], sem_w[s]
        ).wait()


def kernel(indices, tables):
    batch = indices.shape[0]

    idx_t = indices.T
    tab_t = jnp.transpose(tables, (0, 2, 1))

    mesh = plsc.VectorSubcoreMesh(core_axis_name="c", subcore_axis_name="s")
    lookup = functools.partial(
        pl.kernel,
        out_type=jax.ShapeDtypeStruct((TROWS, batch), jnp.float32),
        mesh=mesh,
        scratch_types=[
            pltpu.VMEM((VOCAB,), jnp.float32),
            pltpu.VMEM((batch,), jnp.int32),
            pltpu.VMEM((2, BCH), jnp.float32),
            pltpu.SemaphoreType.DMA,
            pltpu.SemaphoreType.DMA,
            pltpu.SemaphoreType.DMA,
        ],
        compiler_params=pltpu.CompilerParams(
            use_tc_tiling_on_sc=True, needs_layout_passes=False
        ),
    )(_lookup_body)

    out_t = lookup(idx_t, tab_t)
    return out_t.T


# speedup vs baseline: 1.2611x; 1.0007x over previous
import functools

import jax
import jax.numpy as jnp
from jax import lax
from jax.experimental import pallas as pl
from jax.experimental.pallas import tpu as pltpu
from jax.experimental.pallas import tpu_sc as plsc

N_FIELDS = 26
VOCAB = 100000
DIM = 16
NC = 2
NS = 16
NW = NC * NS
TROWS = N_FIELDS * DIM
RPW = TROWS // NW
BCH = 4096
NCH = 4


def _lookup_body(idx_hbm, tab_hbm, out_hbm, trow_v, idx_v, grow_v,
                 sem_t, sem_w0, sem_w1):
    wid = lax.axis_index("s") * NC + lax.axis_index("c")
    sem_w = (sem_w0, sem_w1)

    def row_loop(r, prev_f):
        row = wid * RPW + r
        f = row // DIM
        d = lax.rem(row, DIM)

        pltpu.async_copy(tab_hbm.at[f, d], trow_v, sem_t)

        @pl.when(f != prev_f)
        def _():
            pltpu.sync_copy(idx_hbm.at[f], idx_v)

        pltpu.make_async_copy(tab_hbm.at[f, d], trow_v, sem_t).wait()

        for c in range(NCH):
            s = c % 2
            if c >= 2:
                pltpu.make_async_copy(
                    grow_v.at[s], out_hbm.at[row, pl.ds(0, BCH)], sem_w[s]
                ).wait()
            else:
                @pl.when(r > 0)
                def _():
                    pltpu.make_async_copy(
                        grow_v.at[s], out_hbm.at[row, pl.ds(0, BCH)], sem_w[s]
                    ).wait()

            def g(i, carry3):
                iv = idx_v[pl.ds(c * BCH + i * 16, 16)]
                grow_v[s, pl.ds(i * 16, 16)] = plsc.load_gather(trow_v, [iv])
                return carry3

            lax.fori_loop(0, BCH // 16, g, 0, unroll=16)
            pltpu.async_copy(
                grow_v.at[s], out_hbm.at[row, pl.ds(c * BCH, BCH)], sem_w[s]
            )
        return f

    last_f = lax.fori_loop(0, RPW, row_loop, -1)
    last_row = wid * RPW + RPW - 1
    for s in range(2):
        pltpu.make_async_copy(
            grow_v.at[s], out_hbm.at[last_row, pl.ds(0, BCH)], sem_w[s]
        ).wait()


def kernel(indices, tables):
    batch = indices.shape[0]

    idx_t = indices.T
    tab_t = jnp.transpose(tables, (0, 2, 1))

    mesh = plsc.VectorSubcoreMesh(core_axis_name="c", subcore_axis_name="s")
    lookup = functools.partial(
        pl.kernel,
        out_type=jax.ShapeDtypeStruct((TROWS, batch), jnp.float32),
        mesh=mesh,
        scratch_types=[
            pltpu.VMEM((VOCAB,), jnp.float32),
            pltpu.VMEM((batch,), jnp.int32),
            pltpu.VMEM((2, BCH), jnp.float32),
            pltpu.SemaphoreType.DMA,
            pltpu.SemaphoreType.DMA,
            pltpu.SemaphoreType.DMA,
        ],
        compiler_params=pltpu.CompilerParams(
            use_tc_tiling_on_sc=True, needs_layout_passes=False
        ),
    )(_lookup_body)

    out_t = lookup(idx_t, tab_t)
    return out_t.T
